# Initial kernel scaffold; baseline (speedup 1.0000x reference)
import jax
import jax.numpy as jnp
from jax.experimental import pallas as pl


def kernel(text_feat, text_mask, img_feat):
    raise NotImplementedError("write your pallas kernel here")


# speedup vs baseline: 19.2438x; 19.2438x over previous
import functools

import jax
import jax.numpy as jnp
from jax import lax
from jax.experimental import pallas as pl
from jax.experimental.pallas import tpu as pltpu
from jax.experimental.pallas import tpu_sc as plsc

L = 16
K = 10
B = 8
N = 32768
C = 256
CH = 128
NGT = CH // L
TSEG = 2
NT = N // TSEG
CHUNK = 512
U = 8

NEG_INF = float("-inf")


def _tree_max(vs):
    while len(vs) > 1:
        vs = [jnp.maximum(vs[i], vs[i + 1]) for i in range(0, len(vs) - 1, 2)] + (
            [vs[-1]] if len(vs) % 2 else [])
    return vs[0]


def _insert(cand, thresh, v):
    mj = v > thresh
    replaced = jnp.zeros((L,), jnp.bool_)
    out = []
    for ci in cand:
        take = mj & (ci == thresh) & (~replaced)
        out.append(jnp.where(take, v, ci))
        replaced = replaced | take
    new_thresh = out[0]
    for ci in out[1:]:
        new_thresh = jnp.minimum(new_thresh, ci)
    return tuple(out) + (new_thresh,)


def _oe_sort_desc(vs):
    vs = list(vs)
    n = len(vs)
    for p in range(n):
        for i in range(p % 2, n - 1, 2):
            hi = jnp.maximum(vs[i], vs[i + 1])
            lo = jnp.minimum(vs[i], vs[i + 1])
            vs[i], vs[i + 1] = hi, lo
    return vs


def _scan_phase(text_feat):
    mesh = plsc.VectorSubcoreMesh(core_axis_name="c", subcore_axis_name="s")

    @functools.partial(
        pl.kernel,
        mesh=mesh,
        out_type=jax.ShapeDtypeStruct((TSEG, B, K, C), jnp.float32),
        scratch_types=[
            pltpu.VMEM((CHUNK, CH), jnp.float32),
            pltpu.VMEM((K + 1, CH), jnp.float32),
        ],
        compiler_params=pltpu.CompilerParams(needs_layout_passes=False),
    )
    def k(text_hbm, part_hbm, buf_v, state_v):
        wid = lax.axis_index("s") * 2 + lax.axis_index("c")
        b = wid % B
        half = (wid // B) % 2
        tseg = wid // 16
        c0 = half * CH
        t0 = tseg * NT

        ninf = jnp.full((L,), NEG_INF, jnp.float32)
        for g in range(NGT):
            for i in range(K + 1):
                state_v[i, pl.ds(g * L, L)] = ninf

        def chunk_body(ci, _):
            pltpu.sync_copy(
                text_hbm.at[b, pl.ds(t0 + ci * CHUNK, CHUNK),
                            pl.ds(c0, CH)],
                buf_v)
            for g in range(NGT):
                st = tuple(state_v[i, pl.ds(g * L, L)] for i in range(K + 1))

                def grp_body(ti, st2, g=g):
                    base = ti * U
                    vs = [buf_v[base + j, pl.ds(g * L, L)] for j in range(U)]
                    m = _tree_max(list(vs))
                    thresh = st2[K]

                    def slow(st3):
                        cur = st3
                        for j in range(U):
                            def do_ins(s, vj=vs[j]):
                                return _insert(s[:K], s[K], vj)

                            cur = lax.cond(
                                jnp.any(vs[j] > cur[K]), do_ins,
                                lambda s: s, cur)
                        return cur

                    return lax.cond(jnp.any(m > thresh), slow,
                                    lambda s: s, st2)

                st = lax.fori_loop(0, CHUNK // U, grp_body, st)
                for i in range(K + 1):
                    state_v[i, pl.ds(g * L, L)] = st[i]
            return 0

        lax.fori_loop(0, NT // CHUNK, chunk_body, 0)

        pltpu.sync_copy(state_v.at[pl.ds(0, K)],
                        part_hbm.at[tseg, b, :, pl.ds(c0, CH)])

    return k(text_feat)


def _merge_phase(partial):
    mesh = plsc.VectorSubcoreMesh(core_axis_name="c", subcore_axis_name="s")

    @functools.partial(
        pl.kernel,
        mesh=mesh,
        out_type=jax.ShapeDtypeStruct((B, K, C), jnp.float32),
        scratch_types=[
            pltpu.VMEM((TSEG * K, CH), jnp.float32),
            pltpu.VMEM((K, CH), jnp.float32),
        ],
        compiler_params=pltpu.CompilerParams(needs_layout_passes=False),
    )
    def k(part_hbm, out_hbm, pv, res_v):
        wid = lax.axis_index("s") * 2 + lax.axis_index("c")

        @pl.when(wid < B * 2)
        def _():
            b = wid % B
            half = wid // B
            c0 = half * CH
            for t in range(TSEG):
                pltpu.sync_copy(part_hbm.at[t, b, :, pl.ds(c0, CH)],
                                pv.at[pl.ds(t * K, K)])
            for g in range(NGT):
                vals = [pv[t * K + i, pl.ds(g * L, L)]
                        for t in range(TSEG) for i in range(K)]
                top = _oe_sort_desc(vals)[:K]
                for i in range(K):
                    res_v[i, pl.ds(g * L, L)] = top[i]
            pltpu.sync_copy(res_v, out_hbm.at[b, :, pl.ds(c0, CH)])

    return k(partial)


def kernel(text_feat, text_mask, img_feat):
    del text_mask, img_feat
    return _merge_phase(_scan_phase(text_feat))


# speedup vs baseline: 37.1517x; 1.9306x over previous
import functools

import jax
import jax.numpy as jnp
from jax import lax
from jax.experimental import pallas as pl
from jax.experimental.pallas import tpu as pltpu
from jax.experimental.pallas import tpu_sc as plsc

L = 16
K = 10
B = 8
N = 32768
C = 256
CH = 128
NGT = CH // L
TSEG = 2
NT = N // TSEG
CHUNK = 256
NCHUNK = NT // CHUNK
U = 16
SUB = 4

NEG_INF = float("-inf")


def _tree_max(vs):
    vs = list(vs)
    while len(vs) > 1:
        vs = [jnp.maximum(vs[i], vs[i + 1]) for i in range(0, len(vs) - 1, 2)] + (
            [vs[-1]] if len(vs) % 2 else [])
    return vs[0]


def _sorted_insert(c, v):
    out = []
    m_prev = None
    for i, ci in enumerate(c):
        mi = v > ci
        if i == 0:
            ni = jnp.where(mi, v, ci)
        else:
            ni = jnp.where(mi, jnp.where(m_prev, c[i - 1], v), ci)
        out.append(ni)
        m_prev = mi
    return tuple(out)


def _scan_phase(text_feat):
    mesh = plsc.VectorSubcoreMesh(core_axis_name="c", subcore_axis_name="s")

    @functools.partial(
        pl.kernel,
        mesh=mesh,
        out_type=jax.ShapeDtypeStruct((TSEG, B, K, C), jnp.float32),
        scratch_types=[
            pltpu.VMEM((CHUNK, CH), jnp.float32),
            pltpu.VMEM((CHUNK, CH), jnp.float32),
            pltpu.VMEM((K, CH), jnp.float32),
            pltpu.SemaphoreType.DMA,
            pltpu.SemaphoreType.DMA,
        ],
        compiler_params=pltpu.CompilerParams(needs_layout_passes=False),
    )
    def k(text_hbm, part_hbm, buf_a, buf_b, state_v, sem_a, sem_b):
        wid = lax.axis_index("s") * 2 + lax.axis_index("c")
        b = wid % B
        half = (wid // B) % 2
        tseg = wid // 16
        c0 = half * CH
        t0 = tseg * NT

        def chunk_src(ci):
            return text_hbm.at[b, pl.ds(t0 + ci * CHUNK, CHUNK),
                               pl.ds(c0, CH)]

        ninf = jnp.full((L,), NEG_INF, jnp.float32)
        for g in range(NGT):
            for i in range(K):
                state_v[i, pl.ds(g * L, L)] = ninf

        def process(buf):
            for g in range(NGT):
                st = tuple(state_v[i, pl.ds(g * L, L)] for i in range(K))

                def grp_body(ti, st2, g=g, buf=buf):
                    base = ti * U
                    vs = [buf[base + j, pl.ds(g * L, L)] for j in range(U)]
                    m = _tree_max(vs)

                    def hot(s):
                        for sub in range(U // SUB):
                            sv = vs[sub * SUB:(sub + 1) * SUB]
                            ms = _tree_max(sv)

                            def hot_sub(s2, sv=sv):
                                for v in sv:
                                    s2 = _sorted_insert(s2, v)
                                return s2

                            s = lax.cond(jnp.any(ms > s[K - 1]), hot_sub,
                                         lambda x: x, s)
                        return s

                    return lax.cond(jnp.any(m > st2[K - 1]), hot,
                                    lambda x: x, st2)

                st = lax.fori_loop(0, CHUNK // U, grp_body, st)
                for i in range(K):
                    state_v[i, pl.ds(g * L, L)] = st[i]

        pltpu.async_copy(chunk_src(0), buf_a, sem_a)

        def pair_body(i2, _):
            ca = 2 * i2
            cb = 2 * i2 + 1
            cn = (2 * i2 + 2) % NCHUNK
            pltpu.async_copy(chunk_src(cb), buf_b, sem_b)
            pltpu.make_async_copy(chunk_src(ca), buf_a, sem_a).wait()
            process(buf_a)
            pltpu.async_copy(chunk_src(cn), buf_a, sem_a)
            pltpu.make_async_copy(chunk_src(cb), buf_b, sem_b).wait()
            process(buf_b)
            return 0

        lax.fori_loop(0, NCHUNK // 2, pair_body, 0)
        pltpu.make_async_copy(chunk_src(0), buf_a, sem_a).wait()

        pltpu.sync_copy(state_v, part_hbm.at[tseg, b, :, pl.ds(c0, CH)])

    return k(text_feat)


def _merge_phase(partial):
    mesh = plsc.VectorSubcoreMesh(core_axis_name="c", subcore_axis_name="s")

    @functools.partial(
        pl.kernel,
        mesh=mesh,
        out_type=jax.ShapeDtypeStruct((B, K, C), jnp.float32),
        scratch_types=[
            pltpu.VMEM((TSEG * K, CH), jnp.float32),
            pltpu.VMEM((K, CH), jnp.float32),
        ],
        compiler_params=pltpu.CompilerParams(needs_layout_passes=False),
    )
    def k(part_hbm, out_hbm, pv, res_v):
        wid = lax.axis_index("s") * 2 + lax.axis_index("c")

        @pl.when(wid < B * TSEG)
        def _():
            b = wid % B
            half = wid // B
            c0 = half * CH
            for t in range(TSEG):
                pltpu.sync_copy(part_hbm.at[t, b, :, pl.ds(c0, CH)],
                                pv.at[pl.ds(t * K, K)])
            for g in range(NGT):
                st = tuple(pv[i, pl.ds(g * L, L)] for i in range(K))
                for i in range(K):
                    st = _sorted_insert(st, pv[K + i, pl.ds(g * L, L)])
                for i in range(K):
                    res_v[i, pl.ds(g * L, L)] = st[i]
            pltpu.sync_copy(res_v, out_hbm.at[b, :, pl.ds(c0, CH)])

    return k(partial)


def kernel(text_feat, text_mask, img_feat):
    del text_mask, img_feat
    return _merge_phase(_scan_phase(text_feat))


# speedup vs baseline: 44.8529x; 1.2073x over previous
import functools

import jax
import jax.numpy as jnp
from jax import lax
from jax.experimental import pallas as pl
from jax.experimental.pallas import tpu as pltpu
from jax.experimental.pallas import tpu_sc as plsc

L = 16
K = 10
B = 8
N = 32768
C = 256
CH = 128
NGT = CH // L
TSEG = 2
NT = N // TSEG
CHUNK = 256
NCHUNK = NT // CHUNK
U = 16
SUB = 4

NEG_INF = float("-inf")


def _any(mask):
    return plsc.all_reduce_population_count(mask)[0] > 0


def _tree_max(vs):
    vs = list(vs)
    while len(vs) > 1:
        vs = [jnp.maximum(vs[i], vs[i + 1]) for i in range(0, len(vs) - 1, 2)] + (
            [vs[-1]] if len(vs) % 2 else [])
    return vs[0]


def _sorted_insert(c, v):
    out = [jnp.maximum(v, c[0])]
    for i in range(1, len(c)):
        out.append(jnp.minimum(c[i - 1], jnp.maximum(v, c[i])))
    return tuple(out)


def _scan_phase(text_feat):
    mesh = plsc.VectorSubcoreMesh(core_axis_name="c", subcore_axis_name="s")

    @functools.partial(
        pl.kernel,
        mesh=mesh,
        out_type=jax.ShapeDtypeStruct((TSEG, B, K, C), jnp.float32),
        scratch_types=[
            pltpu.VMEM((CHUNK, CH), jnp.float32),
            pltpu.VMEM((CHUNK, CH), jnp.float32),
            pltpu.VMEM((K, CH), jnp.float32),
            pltpu.SemaphoreType.DMA,
            pltpu.SemaphoreType.DMA,
        ],
        compiler_params=pltpu.CompilerParams(needs_layout_passes=False),
    )
    def k(text_hbm, part_hbm, buf_a, buf_b, state_v, sem_a, sem_b):
        wid = lax.axis_index("s") * 2 + lax.axis_index("c")
        b = wid % B
        half = (wid // B) % 2
        tseg = wid // 16
        c0 = half * CH
        t0 = tseg * NT

        def chunk_src(ci):
            return text_hbm.at[b, pl.ds(t0 + ci * CHUNK, CHUNK),
                               pl.ds(c0, CH)]

        ninf = jnp.full((L,), NEG_INF, jnp.float32)
        for g in range(NGT):
            for i in range(K):
                state_v[i, pl.ds(g * L, L)] = ninf

        def process(buf):
            for g in range(NGT):
                gs = pl.ds(g * L, L)

                def grp_body(ti, thresh, g=g, gs=gs, buf=buf):
                    base = ti * U
                    vs = [buf[base + j, gs] for j in range(U)]
                    subs = [vs[s * SUB:(s + 1) * SUB]
                            for s in range(U // SUB)]
                    submax = [_tree_max(sv) for sv in subs]
                    m = _tree_max(submax)

                    def hot(th, gs=gs):
                        st = tuple(state_v[i, gs] for i in range(K))
                        for sv, ms in zip(subs, submax):

                            def hot_sub(s2, sv=sv):
                                for v in sv:
                                    s2 = _sorted_insert(s2, v)
                                return s2

                            st = lax.cond(_any(ms > st[K - 1]), hot_sub,
                                          lambda x: x, st)
                        for i in range(K):
                            state_v[i, gs] = st[i]
                        return st[K - 1]

                    return lax.cond(_any(m > thresh), hot,
                                    lambda x: x, thresh)

                lax.fori_loop(0, CHUNK // U, grp_body, state_v[K - 1, gs])

        pltpu.async_copy(chunk_src(0), buf_a, sem_a)

        def pair_body(i2, _):
            ca = 2 * i2
            cb = 2 * i2 + 1
            cn = (2 * i2 + 2) % NCHUNK
            pltpu.async_copy(chunk_src(cb), buf_b, sem_b)
            pltpu.make_async_copy(chunk_src(ca), buf_a, sem_a).wait()
            process(buf_a)
            pltpu.async_copy(chunk_src(cn), buf_a, sem_a)
            pltpu.make_async_copy(chunk_src(cb), buf_b, sem_b).wait()
            process(buf_b)
            return 0

        lax.fori_loop(0, NCHUNK // 2, pair_body, 0)
        pltpu.make_async_copy(chunk_src(0), buf_a, sem_a).wait()

        pltpu.sync_copy(state_v, part_hbm.at[tseg, b, :, pl.ds(c0, CH)])

    return k(text_feat)


def _merge_phase(partial):
    mesh = plsc.VectorSubcoreMesh(core_axis_name="c", subcore_axis_name="s")

    @functools.partial(
        pl.kernel,
        mesh=mesh,
        out_type=jax.ShapeDtypeStruct((B, K, C), jnp.float32),
        scratch_types=[
            pltpu.VMEM((TSEG * K, CH), jnp.float32),
            pltpu.VMEM((K, CH), jnp.float32),
        ],
        compiler_params=pltpu.CompilerParams(needs_layout_passes=False),
    )
    def k(part_hbm, out_hbm, pv, res_v):
        wid = lax.axis_index("s") * 2 + lax.axis_index("c")

        @pl.when(wid < B * TSEG)
        def _():
            b = wid % B
            half = wid // B
            c0 = half * CH
            for t in range(TSEG):
                pltpu.sync_copy(part_hbm.at[t, b, :, pl.ds(c0, CH)],
                                pv.at[pl.ds(t * K, K)])
            for g in range(NGT):
                st = tuple(pv[i, pl.ds(g * L, L)] for i in range(K))
                for i in range(K):
                    st = _sorted_insert(st, pv[K + i, pl.ds(g * L, L)])
                for i in range(K):
                    res_v[i, pl.ds(g * L, L)] = st[i]
            pltpu.sync_copy(res_v, out_hbm.at[b, :, pl.ds(c0, CH)])

    return k(partial)


def kernel(text_feat, text_mask, img_feat):
    del text_mask, img_feat
    return _merge_phase(_scan_phase(text_feat))
